# Initial kernel scaffold; baseline (speedup 1.0000x reference)
import jax
import jax.numpy as jnp
from jax.experimental import pallas as pl


def kernel(text, table, W, b):
    raise NotImplementedError("write your pallas kernel here")


# speedup vs baseline: 11.1845x; 11.1845x over previous
import functools

import jax
import jax.numpy as jnp
from jax import lax
from jax.experimental import pallas as pl
from jax.experimental.pallas import tpu as pltpu
from jax.experimental.pallas import tpu_sc as plsc

_VOCAB = 100000
_DIM = 128
_NCLS = 3
_BATCH = 4096
_SEQ = 200
_PAD = 16

_NC = 2
_NS = 16
_NW = _NC * _NS
_BPW = _BATCH // _NW
_TPW = _BPW * _SEQ
_GRP = 2
_CHUNK = _GRP * _SEQ
_NITER = _TPW // _CHUNK
_SPLITS = ((0, 128), (128, 128), (256, 128), (384, 16))
_UNROLL = 8


def _proj_body(t_ref, w_ref, b_ref, o_ref):
    o_ref[...] = (
        jnp.dot(t_ref[...], w_ref[...], preferred_element_type=jnp.float32,
                precision=lax.Precision.HIGHEST)
        + b_ref[...]
    )


def _project(table, w_s, b_s):
    blk = 1000
    return pl.pallas_call(
        _proj_body,
        grid=(_VOCAB // blk,),
        in_specs=[
            pl.BlockSpec((blk, _DIM), lambda i: (i, 0)),
            pl.BlockSpec((_DIM, _PAD), lambda i: (0, 0)),
            pl.BlockSpec((1, _PAD), lambda i: (0, 0)),
        ],
        out_specs=pl.BlockSpec((blk, _PAD), lambda i: (i, 0)),
        out_shape=jax.ShapeDtypeStruct((_VOCAB, _PAD), jnp.float32),
    )(table, w_s, b_s)


_mesh = plsc.VectorSubcoreMesh(core_axis_name="c", subcore_axis_name="s")


@functools.partial(
    pl.kernel,
    mesh=_mesh,
    compiler_params=pltpu.CompilerParams(use_tc_tiling_on_sc=False),
    out_type=jax.ShapeDtypeStruct((_BATCH, _PAD), jnp.float32),
    scratch_types=[
        pltpu.VMEM((_CHUNK,), jnp.int32),
        pltpu.VMEM((_CHUNK,), jnp.int32),
        pltpu.VMEM((_CHUNK, _PAD), jnp.float32),
        pltpu.VMEM((_CHUNK, _PAD), jnp.float32),
        pltpu.VMEM((_BPW, _PAD), jnp.float32),
        pltpu.SemaphoreType.DMA,
        pltpu.SemaphoreType.DMA,
        pltpu.SemaphoreType.DMA,
        pltpu.SemaphoreType.DMA,
    ],
)
def _sc_pool(text_hbm, p_hbm, out_hbm,
             idx0, idx1, rows0, rows1, out_v,
             tsem0, tsem1, gsem0, gsem1):
    wid = lax.axis_index("s") * _NC + lax.axis_index("c")
    tok0 = wid * _TPW
    idxb = (idx0, idx1)
    rowsb = (rows0, rows1)
    tsems = (tsem0, tsem1)
    gsems = (gsem0, gsem1)

    def text_copy(bf, cc):
        return pltpu.make_async_copy(
            text_hbm.at[pl.ds(tok0 + cc * _CHUNK, _CHUNK)], idxb[bf], tsems[bf]
        )

    def fire_gathers(bf):
        for off, n in _SPLITS:
            pltpu.make_async_copy(
                p_hbm.at[idxb[bf].at[pl.ds(off, n)]],
                rowsb[bf].at[pl.ds(off, n)],
                gsems[bf],
            ).start()

    def drain_gathers(bf):
        for off, n in _SPLITS:
            pltpu.make_async_copy(
                p_hbm.at[idxb[bf].at[pl.ds(off, n)]],
                rowsb[bf].at[pl.ds(off, n)],
                gsems[bf],
            ).wait()

    text_copy(0, 0).start()
    text_copy(1, 1).start()
    text_copy(0, 0).wait()
    fire_gathers(0)

    def step(bf, cc):
        drain_gathers(bf)
        nb = 1 - bf

        @pl.when(cc + 1 < _NITER)
        def _():
            text_copy(nb, cc + 1).wait()
            fire_gathers(nb)

        @pl.when(cc + 2 < _NITER)
        def _():
            text_copy(bf, cc + 2).start()

        rows = rowsb[bf]

        def acc_body(j, carry):
            a0, a1 = carry
            base = j * _UNROLL
            for u in range(_UNROLL):
                a0 = a0 + rows[base + u, :]
                a1 = a1 + rows[_SEQ + base + u, :]
            return (a0, a1)

        zero = jnp.zeros((_PAD,), jnp.float32)
        a0, a1 = lax.fori_loop(0, _SEQ // _UNROLL, acc_body, (zero, zero))
        out_v[_GRP * cc, :] = a0
        out_v[_GRP * cc + 1, :] = a1

    def outer(c, carry):
        step(0, 2 * c)
        step(1, 2 * c + 1)
        return carry

    lax.fori_loop(0, _NITER // 2, outer, 0)
    pltpu.sync_copy(out_v, out_hbm.at[pl.ds(wid * _BPW, _BPW)])


def kernel(text, table, W, b):
    inv = jnp.float32(1.0 / _SEQ)
    w_s = jnp.pad(W, ((0, 0), (0, _PAD - _NCLS))) * inv
    b_s = (jnp.pad(b, (0, _PAD - _NCLS)) * inv).reshape(1, _PAD)
    p = _project(table, w_s, b_s)
    textf = text.reshape(-1).astype(jnp.int32)
    out = _sc_pool(textf, p)
    return out[:, :_NCLS]


# speedup vs baseline: 13.8747x; 1.2405x over previous
import functools

import jax
import jax.numpy as jnp
from jax import lax
from jax.experimental import pallas as pl
from jax.experimental.pallas import tpu as pltpu
from jax.experimental.pallas import tpu_sc as plsc

_VOCAB = 100000
_DIM = 128
_NCLS = 3
_BATCH = 4096
_SEQ = 200
_PAD = 16

_NC = 2
_NS = 16
_NW = _NC * _NS
_BPW = _BATCH // _NW
_TPW = _BPW * _SEQ
_GRP = 2
_CHUNK = _GRP * _SEQ
_NITER = _TPW // _CHUNK
_SPLITS = ((0, 128), (128, 128), (256, 128), (384, 16))
_UNROLL = 8


def _proj_body(t_ref, w_ref, b_ref, o_ref):
    o_ref[...] = (
        jnp.dot(t_ref[...], w_ref[...], preferred_element_type=jnp.float32,
                precision=lax.Precision.HIGHEST)
        + b_ref[...]
    )


def _project(table, w_s, b_s):
    blk = 1000
    return pl.pallas_call(
        _proj_body,
        grid=(_VOCAB // blk,),
        in_specs=[
            pl.BlockSpec((blk, _DIM), lambda i: (i, 0)),
            pl.BlockSpec((_DIM, _PAD), lambda i: (0, 0)),
            pl.BlockSpec((1, _PAD), lambda i: (0, 0)),
        ],
        out_specs=pl.BlockSpec((blk, _PAD), lambda i: (i, 0)),
        out_shape=jax.ShapeDtypeStruct((_VOCAB, _PAD), jnp.float32),
    )(table, w_s, b_s)


_mesh = plsc.VectorSubcoreMesh(core_axis_name="c", subcore_axis_name="s")

_SUNROLL = 8


@functools.partial(
    pl.kernel,
    mesh=_mesh,
    compiler_params=pltpu.CompilerParams(use_tc_tiling_on_sc=False),
    out_type=jax.ShapeDtypeStruct((_BATCH, _PAD), jnp.float32),
    scratch_types=[
        pltpu.VMEM((_SEQ, _BPW), jnp.int32),
        pltpu.VMEM((_BPW, _PAD), jnp.float32),
        pltpu.SemaphoreType.DMA,
        pltpu.SemaphoreType.DMA,
    ],
)
def _sc_pool(textT_hbm, p_hbm, out_hbm, slab, acc, tsem, gsem):
    wid = lax.axis_index("s") * _NC + lax.axis_index("c")
    b0 = wid * _BPW

    pltpu.make_async_copy(
        textT_hbm.at[:, pl.ds(b0, _BPW)], slab, tsem
    ).start()

    zero = jnp.zeros((_PAD,), jnp.float32)
    def zbody(i, carry):
        acc[i, :] = zero
        return carry
    lax.fori_loop(0, _BPW, zbody, 0)

    pltpu.make_async_copy(
        textT_hbm.at[:, pl.ds(b0, _BPW)], slab, tsem
    ).wait()

    def fire(j, carry):
        for u in range(_SUNROLL):
            pltpu.make_async_copy(
                p_hbm.at[slab.at[j * _SUNROLL + u]], acc, gsem
            ).start(add=True)
        return carry
    lax.fori_loop(0, _SEQ // _SUNROLL, fire, 0)

    def drain(j, carry):
        for u in range(_SUNROLL):
            pltpu.make_async_copy(
                p_hbm.at[slab.at[j * _SUNROLL + u]], acc, gsem
            ).wait()
        return carry
    lax.fori_loop(0, _SEQ // _SUNROLL, drain, 0)

    pltpu.sync_copy(acc, out_hbm.at[pl.ds(b0, _BPW)])


def kernel(text, table, W, b):
    inv = jnp.float32(1.0 / _SEQ)
    w_s = jnp.pad(W, ((0, 0), (0, _PAD - _NCLS))) * inv
    b_s = (jnp.pad(b, (0, _PAD - _NCLS)) * inv).reshape(1, _PAD)
    p = _project(table, w_s, b_s)
    text_t = text.astype(jnp.int32).T
    out = _sc_pool(text_t, p)
    return out[:, :_NCLS]


# speedup vs baseline: 19.6184x; 1.4140x over previous
import functools

import jax
import jax.numpy as jnp
from jax import lax
from jax.experimental import pallas as pl
from jax.experimental.pallas import tpu as pltpu
from jax.experimental.pallas import tpu_sc as plsc

_VOCAB = 100000
_DIM = 128
_NCLS = 3
_BATCH = 4096
_SEQ = 200
_PAD = 16

_NC = 2
_NS = 16
_NW = _NC * _NS
_BPW = _BATCH // _NW
_TPW = _BPW * _SEQ
_GRP = 2
_CHUNK = _GRP * _SEQ
_NITER = _TPW // _CHUNK
_SPLITS = ((0, 128), (128, 128), (256, 128), (384, 16))
_UNROLL = 8


def _proj_body(t_ref, w_ref, b_ref, o_ref):
    o_ref[...] = (
        jnp.dot(t_ref[...], w_ref[...], preferred_element_type=jnp.float32)
        + b_ref[...]
    )


def _project(table, w_s, b_s):
    blk = 2000
    return pl.pallas_call(
        _proj_body,
        grid=(_VOCAB // blk,),
        in_specs=[
            pl.BlockSpec((blk, _DIM), lambda i: (i, 0)),
            pl.BlockSpec((_DIM, _PAD), lambda i: (0, 0)),
            pl.BlockSpec((1, _PAD), lambda i: (0, 0)),
        ],
        out_specs=pl.BlockSpec((blk, _PAD), lambda i: (i, 0)),
        out_shape=jax.ShapeDtypeStruct((_VOCAB, _PAD), jnp.float32),
    )(table, w_s, b_s)


_mesh = plsc.VectorSubcoreMesh(core_axis_name="c", subcore_axis_name="s")

_SUNROLL = 8


@functools.partial(
    pl.kernel,
    mesh=_mesh,
    compiler_params=pltpu.CompilerParams(use_tc_tiling_on_sc=False),
    out_type=jax.ShapeDtypeStruct((_BATCH, _PAD), jnp.float32),
    scratch_types=[
        pltpu.VMEM((_SEQ, _BPW), jnp.int32),
        pltpu.VMEM((_BPW, _PAD), jnp.float32),
        pltpu.SemaphoreType.DMA,
        pltpu.SemaphoreType.DMA,
    ],
)
def _sc_pool(textT_hbm, p_hbm, out_hbm, slab, acc, tsem, gsem):
    wid = lax.axis_index("s") * _NC + lax.axis_index("c")
    b0 = wid * _BPW

    pltpu.make_async_copy(
        textT_hbm.at[:, pl.ds(b0, _BPW)], slab, tsem
    ).start()

    zero = jnp.zeros((_PAD,), jnp.float32)
    def zbody(i, carry):
        acc[i, :] = zero
        return carry
    lax.fori_loop(0, _BPW, zbody, 0)

    pltpu.make_async_copy(
        textT_hbm.at[:, pl.ds(b0, _BPW)], slab, tsem
    ).wait()

    def fire(j, carry):
        for u in range(_SUNROLL):
            pltpu.make_async_copy(
                p_hbm.at[slab.at[j * _SUNROLL + u]], acc, gsem
            ).start(add=True)
        return carry
    lax.fori_loop(0, _SEQ // _SUNROLL, fire, 0)

    def drain(j, carry):
        for u in range(_SUNROLL):
            pltpu.make_async_copy(
                p_hbm.at[slab.at[j * _SUNROLL + u]], acc, gsem
            ).wait()
        return carry
    lax.fori_loop(0, _SEQ // _SUNROLL, drain, 0)

    pltpu.sync_copy(acc, out_hbm.at[pl.ds(b0, _BPW)])


def kernel(text, table, W, b):
    inv = jnp.float32(1.0 / _SEQ)
    w_s = jnp.pad(W, ((0, 0), (0, _PAD - _NCLS))) * inv
    b_s = (jnp.pad(b, (0, _PAD - _NCLS)) * inv).reshape(1, _PAD)
    p = _project(table, w_s, b_s)
    text_t = text.astype(jnp.int32).T
    out = _sc_pool(text_t, p)
    return out[:, :_NCLS]


# speedup vs baseline: 22.3052x; 1.1370x over previous
import functools

import jax
import jax.numpy as jnp
from jax import lax
from jax.experimental import pallas as pl
from jax.experimental.pallas import tpu as pltpu
from jax.experimental.pallas import tpu_sc as plsc

_VOCAB = 100000
_DIM = 128
_NCLS = 3
_BATCH = 4096
_SEQ = 200
_PAD = 16

_NC = 2
_NS = 16
_NW = _NC * _NS
_BPW = _BATCH // _NW
_TPW = _BPW * _SEQ
_GRP = 2
_CHUNK = _GRP * _SEQ
_NITER = _TPW // _CHUNK
_SPLITS = ((0, 128), (128, 128), (256, 128), (384, 16))
_UNROLL = 8


def _proj_body(t_ref, w_ref, b_ref, o_ref):
    o_ref[...] = (
        jnp.dot(t_ref[...], w_ref[...], preferred_element_type=jnp.float32)
        + b_ref[...]
    )


def _project(table, w_s, b_s):
    blk = 4000
    return pl.pallas_call(
        _proj_body,
        grid=(_VOCAB // blk,),
        in_specs=[
            pl.BlockSpec((blk, _DIM), lambda i: (i, 0)),
            pl.BlockSpec((_DIM, _PAD), lambda i: (0, 0)),
            pl.BlockSpec((1, _PAD), lambda i: (0, 0)),
        ],
        out_specs=pl.BlockSpec((blk, _PAD), lambda i: (i, 0)),
        out_shape=jax.ShapeDtypeStruct((_VOCAB, _PAD), jnp.float32),
    )(table, w_s, b_s)


_mesh = plsc.VectorSubcoreMesh(core_axis_name="c", subcore_axis_name="s")

_SUNROLL = 8


@functools.partial(
    pl.kernel,
    mesh=_mesh,
    compiler_params=pltpu.CompilerParams(use_tc_tiling_on_sc=False),
    out_type=jax.ShapeDtypeStruct((_BATCH, _PAD), jnp.float32),
    scratch_types=[
        pltpu.VMEM((_SEQ, _BPW), jnp.int32),
        pltpu.VMEM((_BPW, _PAD), jnp.float32),
        pltpu.SemaphoreType.DMA,
        pltpu.SemaphoreType.DMA,
    ],
)
def _sc_pool(textT_hbm, p_hbm, out_hbm, slab, acc, tsem, gsem):
    wid = lax.axis_index("s") * _NC + lax.axis_index("c")
    b0 = wid * _BPW

    pltpu.make_async_copy(
        textT_hbm.at[:, pl.ds(b0, _BPW)], slab, tsem
    ).start()

    zero = jnp.zeros((_PAD,), jnp.float32)
    def zbody(i, carry):
        acc[i, :] = zero
        return carry
    lax.fori_loop(0, _BPW, zbody, 0)

    pltpu.make_async_copy(
        textT_hbm.at[:, pl.ds(b0, _BPW)], slab, tsem
    ).wait()

    def fire(j, carry):
        for u in range(_SUNROLL):
            pltpu.make_async_copy(
                p_hbm.at[slab.at[j * _SUNROLL + u]], acc, gsem
            ).start(add=True)
        return carry
    lax.fori_loop(0, _SEQ // _SUNROLL, fire, 0)

    def drain(j, carry):
        for u in range(_SUNROLL):
            pltpu.make_async_copy(
                p_hbm.at[slab.at[j * _SUNROLL + u]], acc, gsem
            ).wait()
        return carry
    lax.fori_loop(0, _SEQ // _SUNROLL, drain, 0)

    pltpu.sync_copy(acc, out_hbm.at[pl.ds(b0, _BPW)])


def kernel(text, table, W, b):
    inv = jnp.float32(1.0 / _SEQ)
    w_s = jnp.pad(W, ((0, 0), (0, _PAD - _NCLS))) * inv
    b_s = (jnp.pad(b, (0, _PAD - _NCLS)) * inv).reshape(1, _PAD)
    p = _project(table, w_s, b_s)
    text_t = text.astype(jnp.int32).T
    out = _sc_pool(text_t, p)
    return out[:, :_NCLS]


# speedup vs baseline: 23.2703x; 1.0433x over previous
import functools

import jax
import jax.numpy as jnp
from jax import lax
from jax.experimental import pallas as pl
from jax.experimental.pallas import tpu as pltpu
from jax.experimental.pallas import tpu_sc as plsc

_VOCAB = 100000
_DIM = 128
_NCLS = 3
_BATCH = 4096
_SEQ = 200
_PAD = 16

_NC = 2
_NS = 16
_NW = _NC * _NS
_BPW = _BATCH // _NW
_TPW = _BPW * _SEQ
_GRP = 2
_CHUNK = _GRP * _SEQ
_NITER = _TPW // _CHUNK
_SPLITS = ((0, 128), (128, 128), (256, 128), (384, 16))
_UNROLL = 8


def _proj_body(t_ref, w_ref, b_ref, o_ref):
    o_ref[...] = (
        jnp.dot(t_ref[...], w_ref[...], preferred_element_type=jnp.float32)
        + b_ref[...]
    )


def _project(table, w_s, b_s):
    blk = 10000
    return pl.pallas_call(
        _proj_body,
        grid=(_VOCAB // blk,),
        in_specs=[
            pl.BlockSpec((blk, _DIM), lambda i: (i, 0)),
            pl.BlockSpec((_DIM, _PAD), lambda i: (0, 0)),
            pl.BlockSpec((1, _PAD), lambda i: (0, 0)),
        ],
        out_specs=pl.BlockSpec((blk, _PAD), lambda i: (i, 0)),
        out_shape=jax.ShapeDtypeStruct((_VOCAB, _PAD), jnp.float32),
    )(table, w_s, b_s)


_mesh = plsc.VectorSubcoreMesh(core_axis_name="c", subcore_axis_name="s")

_SUNROLL = 8


@functools.partial(
    pl.kernel,
    mesh=_mesh,
    compiler_params=pltpu.CompilerParams(use_tc_tiling_on_sc=False),
    out_type=jax.ShapeDtypeStruct((_BATCH, _PAD), jnp.float32),
    scratch_types=[
        pltpu.VMEM((_SEQ, _BPW), jnp.int32),
        pltpu.VMEM((_BPW, _PAD), jnp.float32),
        pltpu.SemaphoreType.DMA,
        pltpu.SemaphoreType.DMA,
    ],
)
def _sc_pool(textT_hbm, p_hbm, out_hbm, slab, acc, tsem, gsem):
    wid = lax.axis_index("s") * _NC + lax.axis_index("c")
    b0 = wid * _BPW

    pltpu.make_async_copy(
        textT_hbm.at[:, pl.ds(b0, _BPW)], slab, tsem
    ).start()

    zero = jnp.zeros((_PAD,), jnp.float32)
    def zbody(i, carry):
        acc[i, :] = zero
        return carry
    lax.fori_loop(0, _BPW, zbody, 0)

    pltpu.make_async_copy(
        textT_hbm.at[:, pl.ds(b0, _BPW)], slab, tsem
    ).wait()

    def fire(j, carry):
        for u in range(_SUNROLL):
            pltpu.make_async_copy(
                p_hbm.at[slab.at[j * _SUNROLL + u]], acc, gsem
            ).start(add=True)
        return carry
    lax.fori_loop(0, _SEQ // _SUNROLL, fire, 0)

    def drain(j, carry):
        for u in range(_SUNROLL):
            pltpu.make_async_copy(
                p_hbm.at[slab.at[j * _SUNROLL + u]], acc, gsem
            ).wait()
        return carry
    lax.fori_loop(0, _SEQ // _SUNROLL, drain, 0)

    pltpu.sync_copy(acc, out_hbm.at[pl.ds(b0, _BPW)])


def kernel(text, table, W, b):
    inv = jnp.float32(1.0 / _SEQ)
    w_s = jnp.pad(W, ((0, 0), (0, _PAD - _NCLS))) * inv
    b_s = (jnp.pad(b, (0, _PAD - _NCLS)) * inv).reshape(1, _PAD)
    p = _project(table, w_s, b_s)
    text_t = text.astype(jnp.int32).T
    out = _sc_pool(text_t, p)
    return out[:, :_NCLS]
